# baseline (speedup 1.0000x reference)
import functools

import jax
import jax.numpy as jnp
from jax import lax
from jax.experimental import pallas as pl
from jax.experimental.pallas import tpu as pltpu
from jax.experimental.pallas import tpu_sc as plsc

B = 16384
EMBED = 64

NC, NS = 2, 16
NW = NC * NS
B_PER_W = B // NW
IDX_CHUNK = 128
N_CHUNK = B_PER_W // IDX_CHUNK


@functools.cache
def _make_sc_gather():
    mesh = plsc.VectorSubcoreMesh(
        core_axis_name="c", subcore_axis_name="s",
        num_cores=NC, num_subcores=NS)

    @functools.partial(
        pl.kernel,
        mesh=mesh,
        compiler_params=pltpu.CompilerParams(use_tc_tiling_on_sc=False),
        out_type=[
            jax.ShapeDtypeStruct((B, EMBED), jnp.float32),
            jax.ShapeDtypeStruct((B, EMBED), jnp.float32),
        ],
        scratch_types=[
            pltpu.VMEM((N_CHUNK, IDX_CHUNK), jnp.int32),
            pltpu.VMEM((N_CHUNK, IDX_CHUNK), jnp.int32),
            pltpu.VMEM((B_PER_W, EMBED), jnp.float32),
            pltpu.VMEM((B_PER_W, EMBED), jnp.float32),
            pltpu.SemaphoreType.DMA,
            pltpu.SemaphoreType.DMA,
        ],
    )
    def _sc_gather(user_idx, item_idx, user_table, item_table,
                   user_out, item_out, uidx_v, iidx_v, urows_v, irows_v,
                   sem_u, sem_i):
        wid = lax.axis_index("s") * NC + lax.axis_index("c")
        base = wid * B_PER_W
        row0 = wid * N_CHUNK
        pltpu.sync_copy(user_idx.at[pl.ds(row0, N_CHUNK)], uidx_v)
        pltpu.sync_copy(item_idx.at[pl.ds(row0, N_CHUNK)], iidx_v)
        copies = []
        for j in range(N_CHUNK):
            copies.append(pltpu.async_copy(
                user_table.at[uidx_v.at[j]],
                urows_v.at[pl.ds(j * IDX_CHUNK, IDX_CHUNK)], sem_u))
            copies.append(pltpu.async_copy(
                item_table.at[iidx_v.at[j]],
                irows_v.at[pl.ds(j * IDX_CHUNK, IDX_CHUNK)], sem_i))
        for c in copies:
            c.wait()
        pltpu.sync_copy(urows_v, user_out.at[pl.ds(base, B_PER_W)])
        pltpu.sync_copy(irows_v, item_out.at[pl.ds(base, B_PER_W)])

    return _sc_gather


def _mlp_body(u_ref, v_ref, w1u_ref, w1v_ref, b1_ref, w2_ref, b2_ref,
              w3_ref, b3_ref, out_ref):
    h1 = jnp.dot(u_ref[...], w1u_ref[...], preferred_element_type=jnp.float32)
    h1 += jnp.dot(v_ref[...], w1v_ref[...], preferred_element_type=jnp.float32)
    h1 = jnp.maximum(h1 + b1_ref[...], 0.0)
    h2 = jnp.dot(h1, w2_ref[...], preferred_element_type=jnp.float32)
    h2 = jnp.maximum(h2 + b2_ref[...], 0.0)
    logit = jnp.sum(h2 * w3_ref[...], axis=1, keepdims=True) + b3_ref[...]
    out_ref[...] = 5.0 / (1.0 + jnp.exp(-logit))


def _tc_mlp(user_vec, item_vec, W1, b1, W2, b2, W3, b3):
    blk = 2048
    grid = (B // blk,)
    full = lambda shape: pl.BlockSpec(shape, lambda i: (0, 0))
    return pl.pallas_call(
        _mlp_body,
        grid=grid,
        in_specs=[
            pl.BlockSpec((blk, EMBED), lambda i: (i, 0)),
            pl.BlockSpec((blk, EMBED), lambda i: (i, 0)),
            full((EMBED, 128)),
            full((EMBED, 128)),
            full((1, 128)),
            full((128, 64)),
            full((1, 64)),
            full((1, 64)),
            full((1, 1)),
        ],
        out_specs=pl.BlockSpec((blk, 1), lambda i: (i, 0)),
        out_shape=jax.ShapeDtypeStruct((B, 1), jnp.float32),
    )(user_vec, item_vec, W1[:EMBED], W1[EMBED:], b1.reshape(1, -1),
      W2, b2.reshape(1, -1), W3.reshape(1, -1), b3.reshape(1, 1))


def kernel(user_input, item_input, user_table, item_table, W1, b1, W2, b2, W3, b3):
    user_vec, item_vec = _make_sc_gather()(
        user_input.reshape(B // IDX_CHUNK, IDX_CHUNK),
        item_input.reshape(B // IDX_CHUNK, IDX_CHUNK),
        user_table, item_table)
    return _tc_mlp(user_vec, item_vec, W1, b1, W2, b2, W3, b3)


# speedup vs baseline: 1.5780x; 1.5780x over previous
import functools

import jax
import jax.numpy as jnp
from jax import lax
from jax.experimental import pallas as pl
from jax.experimental.pallas import tpu as pltpu
from jax.experimental.pallas import tpu_sc as plsc

B = 16384
EMBED = 64

NC, NS = 2, 16
NW = NC * NS
B_PER_W = B // NW


@functools.cache
def _make_sc_gather():
    mesh = plsc.VectorSubcoreMesh(
        core_axis_name="c", subcore_axis_name="s",
        num_cores=NC, num_subcores=NS)

    @functools.partial(
        pl.kernel,
        mesh=mesh,
        out_type=[
            jax.ShapeDtypeStruct((B, EMBED), jnp.float32),
            jax.ShapeDtypeStruct((B, EMBED), jnp.float32),
        ],
        scratch_types=[
            pltpu.VMEM((B_PER_W,), jnp.int32),
            pltpu.VMEM((B_PER_W, EMBED), jnp.float32),
            pltpu.SemaphoreType.DMA,
        ],
    )
    def _sc_gather(user_idx, item_idx, user_tab, item_tab,
                   user_out, item_out, idx_v, rowsbuf, sem):
        wid = lax.axis_index("s") * NC + lax.axis_index("c")
        base = wid * B_PER_W
        for idx_hbm, tab, out in ((user_idx, user_tab, user_out),
                                  (item_idx, item_tab, item_out)):
            pltpu.sync_copy(idx_hbm.at[pl.ds(base, B_PER_W)], idx_v)

            def body(j, _):
                k0 = j * 16
                v = idx_v[pl.ds(k0, 16)]
                for l in range(16):
                    pltpu.async_copy(
                        tab.at[v[l]], rowsbuf.at[k0 + l], sem)
                return 0

            lax.fori_loop(0, B_PER_W // 16, body, 0)
            pltpu.make_async_copy(
                tab.at[pl.ds(0, B_PER_W)], rowsbuf, sem).wait()
            pltpu.sync_copy(rowsbuf, out.at[pl.ds(base, B_PER_W)])

    return _sc_gather


def _mlp_body(u_ref, v_ref, w1u_ref, w1v_ref, b1_ref, w2_ref, b2_ref,
              w3_ref, b3_ref, out_ref):
    h1 = jnp.dot(u_ref[...], w1u_ref[...], preferred_element_type=jnp.float32)
    h1 += jnp.dot(v_ref[...], w1v_ref[...], preferred_element_type=jnp.float32)
    h1 = jnp.maximum(h1 + b1_ref[...], 0.0)
    h2 = jnp.dot(h1, w2_ref[...], preferred_element_type=jnp.float32)
    h2 = jnp.maximum(h2 + b2_ref[...], 0.0)
    logit = jnp.sum(h2 * w3_ref[...], axis=1, keepdims=True) + b3_ref[...]
    out_ref[...] = 5.0 / (1.0 + jnp.exp(-logit))


def _tc_mlp(user_vec, item_vec, W1, b1, W2, b2, W3, b3):
    blk = 2048
    grid = (B // blk,)
    full = lambda shape: pl.BlockSpec(shape, lambda i: (0, 0))
    return pl.pallas_call(
        _mlp_body,
        grid=grid,
        in_specs=[
            pl.BlockSpec((blk, EMBED), lambda i: (i, 0)),
            pl.BlockSpec((blk, EMBED), lambda i: (i, 0)),
            full((EMBED, 128)),
            full((EMBED, 128)),
            full((1, 128)),
            full((128, 64)),
            full((1, 64)),
            full((1, 64)),
            full((1, 1)),
        ],
        out_specs=pl.BlockSpec((blk, 1), lambda i: (i, 0)),
        out_shape=jax.ShapeDtypeStruct((B, 1), jnp.float32),
    )(user_vec, item_vec, W1[:EMBED], W1[EMBED:], b1.reshape(1, -1),
      W2, b2.reshape(1, -1), W3.reshape(1, -1), b3.reshape(1, 1))


def kernel(user_input, item_input, user_table, item_table, W1, b1, W2, b2, W3, b3):
    user_vec, item_vec = _make_sc_gather()(
        user_input, item_input, user_table, item_table)
    return _tc_mlp(user_vec, item_vec, W1, b1, W2, b2, W3, b3)


# speedup vs baseline: 1.5996x; 1.0137x over previous
import functools

import jax
import jax.numpy as jnp
from jax import lax
from jax.experimental import pallas as pl
from jax.experimental.pallas import tpu as pltpu
from jax.experimental.pallas import tpu_sc as plsc

B = 16384
EMBED = 64

NC, NS = 2, 16
NW = NC * NS
B_PER_W = B // NW


@functools.cache
def _make_sc_gather():
    mesh = plsc.VectorSubcoreMesh(
        core_axis_name="c", subcore_axis_name="s",
        num_cores=NC, num_subcores=NS)

    @functools.partial(
        pl.kernel,
        mesh=mesh,
        out_type=jax.ShapeDtypeStruct((B, 2 * EMBED), jnp.float32),
        scratch_types=[
            pltpu.VMEM((B_PER_W,), jnp.int32),
            pltpu.VMEM((B_PER_W,), jnp.int32),
            pltpu.VMEM((B_PER_W, 2 * EMBED), jnp.float32),
            pltpu.SemaphoreType.DMA,
        ],
    )
    def _sc_gather(user_idx, item_idx, user_tab, item_tab,
                   out, uidx_v, iidx_v, rowsbuf, sem):
        wid = lax.axis_index("s") * NC + lax.axis_index("c")
        base = wid * B_PER_W
        pltpu.sync_copy(user_idx.at[pl.ds(base, B_PER_W)], uidx_v)
        pltpu.sync_copy(item_idx.at[pl.ds(base, B_PER_W)], iidx_v)

        def body(j, _):
            k0 = j * 16
            uv = uidx_v[pl.ds(k0, 16)]
            iv = iidx_v[pl.ds(k0, 16)]
            for l in range(16):
                pltpu.async_copy(
                    user_tab.at[uv[l]],
                    rowsbuf.at[k0 + l, pl.ds(0, EMBED)], sem)
                pltpu.async_copy(
                    item_tab.at[iv[l]],
                    rowsbuf.at[k0 + l, pl.ds(EMBED, EMBED)], sem)
            return 0

        lax.fori_loop(0, B_PER_W // 16, body, 0)
        pltpu.make_async_copy(out.at[pl.ds(0, B_PER_W)], rowsbuf, sem).wait()
        pltpu.sync_copy(rowsbuf, out.at[pl.ds(base, B_PER_W)])

    return _sc_gather


def _mlp_body(x_ref, w1_ref, b1_ref, w2_ref, b2_ref, w3_ref, b3_ref, out_ref):
    h1 = jnp.dot(x_ref[...], w1_ref[...], preferred_element_type=jnp.float32)
    h1 = jnp.maximum(h1 + b1_ref[...], 0.0)
    h2 = jnp.dot(h1, w2_ref[...], preferred_element_type=jnp.float32)
    h2 = jnp.maximum(h2 + b2_ref[...], 0.0)
    logit = jnp.sum(h2 * w3_ref[...], axis=1, keepdims=True) + b3_ref[...]
    out_ref[...] = 5.0 / (1.0 + jnp.exp(-logit))


def _tc_mlp(x, W1, b1, W2, b2, W3, b3):
    blk = 2048
    grid = (B // blk,)
    full = lambda shape: pl.BlockSpec(shape, lambda i: (0, 0))
    return pl.pallas_call(
        _mlp_body,
        grid=grid,
        in_specs=[
            pl.BlockSpec((blk, 2 * EMBED), lambda i: (i, 0)),
            full((2 * EMBED, 128)),
            full((1, 128)),
            full((128, 64)),
            full((1, 64)),
            full((1, 64)),
            full((1, 1)),
        ],
        out_specs=pl.BlockSpec((blk, 1), lambda i: (i, 0)),
        out_shape=jax.ShapeDtypeStruct((B, 1), jnp.float32),
    )(x, W1, b1.reshape(1, -1), W2, b2.reshape(1, -1),
      W3.reshape(1, -1), b3.reshape(1, 1))


def kernel(user_input, item_input, user_table, item_table, W1, b1, W2, b2, W3, b3):
    x = _make_sc_gather()(user_input, item_input, user_table, item_table)
    return _tc_mlp(x, W1, b1, W2, b2, W3, b3)
